# baseline (speedup 1.0000x reference)
import functools

import jax
import jax.numpy as jnp
from jax import lax
from jax.experimental import pallas as pl
from jax.experimental.pallas import tpu as pltpu
from jax.experimental.pallas import tpu_sc as plsc

B = 16384
D = 64
NC = 2
NS = 16
L = 16
NW = NC * NS
BPW = B // NW
CHUNK = 128
NCH = BPW // CHUNK


def _fm_body(uid_hbm, iid_hbm, ut_hbm, it_hbm, ub_hbm, ib_hbm, out_hbm,
             uidx_v, iidx_v, urows_v, irows_v, ubias_v, ibias_v,
             out_v, sem):
    wid = lax.axis_index("s") * NC + lax.axis_index("c")
    base = wid * BPW
    lane = lax.iota(jnp.int32, L)

    pltpu.sync_copy(uid_hbm.at[pl.ds(wid * NCH, NCH)], uidx_v)
    pltpu.sync_copy(iid_hbm.at[pl.ds(wid * NCH, NCH)], iidx_v)

    copies = []
    for j in range(NCH):
        copies.append(pltpu.async_copy(
            ut_hbm.at[uidx_v.at[j]], urows_v.at[pl.ds(j * CHUNK, CHUNK)], sem))
        copies.append(pltpu.async_copy(
            it_hbm.at[iidx_v.at[j]], irows_v.at[pl.ds(j * CHUNK, CHUNK)], sem))
        copies.append(pltpu.async_copy(
            ub_hbm.at[uidx_v.at[j]], ubias_v.at[pl.ds(j * CHUNK, CHUNK)], sem))
        copies.append(pltpu.async_copy(
            ib_hbm.at[iidx_v.at[j]], ibias_v.at[pl.ds(j * CHUNK, CHUNK)], sem))
    for c in copies:
        c.wait()

    def group(g, _):
        dots = jnp.zeros((L,), jnp.float32)
        for r in range(L):
            row = g * L + r
            s = urows_v[row, pl.ds(0, L)] * irows_v[row, pl.ds(0, L)]
            for c in range(1, D // L):
                s = s + urows_v[row, pl.ds(c * L, L)] * irows_v[row, pl.ds(c * L, L)]
            dots = jnp.where(lane == r, jnp.sum(s), dots)
        blk = pl.ds(g * L, L)
        out_v[blk] = dots + ubias_v[blk] + ibias_v[blk]
        return ()

    lax.fori_loop(0, BPW // L, group, (), unroll=False)

    pltpu.sync_copy(out_v, out_hbm.at[pl.ds(base, BPW)])


@functools.partial(
    pl.kernel,
    out_type=jax.ShapeDtypeStruct((B,), jnp.float32),
    mesh=plsc.VectorSubcoreMesh(core_axis_name="c", subcore_axis_name="s"),
    compiler_params=pltpu.CompilerParams(
        needs_layout_passes=False, use_tc_tiling_on_sc=False),
    scratch_types=[
        pltpu.VMEM((NCH, CHUNK), jnp.int32),
        pltpu.VMEM((NCH, CHUNK), jnp.int32),
        pltpu.VMEM((BPW, D), jnp.float32),
        pltpu.VMEM((BPW, D), jnp.float32),
        pltpu.VMEM((BPW,), jnp.float32),
        pltpu.VMEM((BPW,), jnp.float32),
        pltpu.VMEM((BPW,), jnp.float32),
        pltpu.SemaphoreType.DMA,
    ],
)
def _fm(uid_hbm, iid_hbm, ut_hbm, it_hbm, ub_hbm, ib_hbm, out_hbm, *scratch):
    _fm_body(uid_hbm, iid_hbm, ut_hbm, it_hbm, ub_hbm, ib_hbm, out_hbm, *scratch)


def kernel(inputs, user_emb_table, item_emb_table, user_bias_table, item_bias_table):
    idx = inputs.astype(jnp.int32)
    uid = idx[:, 0].reshape(NW * NCH, CHUNK)
    iid = idx[:, 1].reshape(NW * NCH, CHUNK)
    out = _fm(uid, iid, user_emb_table, item_emb_table,
              user_bias_table.reshape(-1), item_bias_table.reshape(-1))
    return out.reshape(B, 1)
